# baseline (speedup 1.0000x reference)
import functools

import jax
import jax.numpy as jnp
from jax import lax
from jax.experimental import pallas as pl
from jax.experimental.pallas import tpu as pltpu
from jax.experimental.pallas import tpu_sc as plsc

_D = 100
_DP = 128
_T = 200
_NB = 4096
_B = _NB * _T
_NW = 32
_WB = _NB // _NW
_HB = _WB // 2

_mesh = plsc.VectorSubcoreMesh(core_axis_name="c", subcore_axis_name="s")


def _pad_body(t_ref, o_ref):
    o_ref[:, :_D] = t_ref[...]
    o_ref[:, _D:] = jnp.zeros((t_ref.shape[0], _DP - _D), jnp.float32)


def _pad_table(t):
    rows, blk = t.shape[0], 2000
    return pl.pallas_call(
        _pad_body,
        grid=(rows // blk,),
        in_specs=[pl.BlockSpec((blk, _D), lambda i: (i, 0))],
        out_specs=pl.BlockSpec((blk, _DP), lambda i: (i, 0)),
        out_shape=jax.ShapeDtypeStruct((rows, _DP), jnp.float32),
    )(t)


@functools.partial(
    pl.kernel,
    out_type=jax.ShapeDtypeStruct((_NB, _T, _D), jnp.float32),
    mesh=_mesh,
    compiler_params=pltpu.CompilerParams(use_tc_tiling_on_sc=True),
    scratch_types=[
        pltpu.VMEM((_HB, _T), jnp.int32),
        pltpu.VMEM((2 * _T, _DP), jnp.float32),
        pltpu.VMEM((2, _T, _D), jnp.float32),
        pltpu.SemaphoreType.DMA,
    ],
)
def _emb_lookup(x_hbm, table_hbm, out_hbm, idx_v, rows_v, comp_v, sem):
    wid = lax.axis_index("s") * 2 + lax.axis_index("c")
    wb = wid * _WB

    for h in range(2):
        pltpu.sync_copy(x_hbm.at[pl.ds(wb + h * _HB, _HB)], idx_v)
        out_base = wb + h * _HB

        def body(it, carry):
            copies = []
            for j in range(2):
                r = 2 * it + j
                copies.append(pltpu.async_copy(
                    table_hbm.at[idx_v.at[r, pl.ds(0, 128)]],
                    rows_v.at[pl.ds(_T * j, 128)], sem))
                copies.append(pltpu.async_copy(
                    table_hbm.at[idx_v.at[r, pl.ds(128, _T - 128)]],
                    rows_v.at[pl.ds(_T * j + 128, _T - 128)], sem))
            for cp in copies:
                cp.wait()

            for j in range(2):
                def row_body(t, c2, j=j):
                    for o in (0, 16, 32, 48, 64, 80, 84):
                        comp_v[j, t, pl.ds(o, 16)] = rows_v[_T * j + t, pl.ds(o, 16)]
                    return c2

                lax.fori_loop(0, _T, row_body, 0)
            pltpu.sync_copy(
                comp_v, out_hbm.at[pl.ds(out_base + it * 2, 2)])
            return carry

        lax.fori_loop(0, _HB // 2, body, 0)


def kernel(x, table):
    table_p = _pad_table(table)
    return _emb_lookup(x.astype(jnp.int32), table_p)


# speedup vs baseline: 1.1709x; 1.1709x over previous
import functools

import jax
import jax.numpy as jnp
from jax import lax
from jax.experimental import pallas as pl
from jax.experimental.pallas import tpu as pltpu
from jax.experimental.pallas import tpu_sc as plsc

_D = 100
_DP = 128
_T = 200
_NB = 4096
_NW = 32
_WB = _NB // _NW
_HB = _WB // 2

_mesh = plsc.VectorSubcoreMesh(core_axis_name="c", subcore_axis_name="s")


def _pad_body(t_ref, o_ref):
    o_ref[:, :_D] = t_ref[...]
    o_ref[:, _D:] = jnp.zeros((t_ref.shape[0], _DP - _D), jnp.float32)


def _pad_table(t):
    rows, blk = t.shape[0], 2000
    return pl.pallas_call(
        _pad_body,
        grid=(rows // blk,),
        in_specs=[pl.BlockSpec((blk, _D), lambda i: (i, 0))],
        out_specs=pl.BlockSpec((blk, _DP), lambda i: (i, 0)),
        out_shape=jax.ShapeDtypeStruct((rows, _DP), jnp.float32),
    )(t)


@functools.partial(
    pl.kernel,
    out_type=jax.ShapeDtypeStruct((_NB, _T, _D), jnp.float32),
    mesh=_mesh,
    compiler_params=pltpu.CompilerParams(use_tc_tiling_on_sc=True),
    scratch_types=[
        pltpu.VMEM((_HB, _T), jnp.int32),
        pltpu.VMEM((_T, _DP), jnp.float32),
        pltpu.VMEM((_T, _DP), jnp.float32),
        pltpu.VMEM((1, _T, _D), jnp.float32),
        pltpu.VMEM((1, _T, _D), jnp.float32),
        pltpu.SemaphoreType.DMA,
        pltpu.SemaphoreType.DMA,
        pltpu.SemaphoreType.DMA,
        pltpu.SemaphoreType.DMA,
    ],
)
def _emb_lookup(x_hbm, table_hbm, out_hbm, idx_v, rows_a, rows_b,
                comp_a, comp_b, sga, sgb, soa, sob):
    wid = lax.axis_index("s") * 2 + lax.axis_index("c")
    wb = wid * _WB

    def fire(r, rows, sg):
        pltpu.async_copy(table_hbm.at[idx_v.at[r, pl.ds(0, 128)]],
                         rows.at[pl.ds(0, 128)], sg)
        pltpu.async_copy(table_hbm.at[idx_v.at[r, pl.ds(128, _T - 128)]],
                         rows.at[pl.ds(128, _T - 128)], sg)

    def wait_gather(rows, sg):
        pltpu.make_async_copy(table_hbm.at[pl.ds(0, _T)], rows, sg).wait()

    def compact(rows, comp):
        def row_body(t, c2):
            for o in (0, 16, 32, 48, 64, 80, 84):
                comp[0, t, pl.ds(o, 16)] = rows[t, pl.ds(o, 16)]
            return c2
        lax.fori_loop(0, _T, row_body, 0)

    def wait_out(comp, so):
        pltpu.make_async_copy(comp, out_hbm.at[pl.ds(0, 1)], so).wait()

    for h in range(2):
        pltpu.sync_copy(x_hbm.at[pl.ds(wb + h * _HB, _HB)], idx_v)
        ob = wb + h * _HB
        fire(0, rows_a, sga)

        def body(i, carry):
            it = 2 * i

            @pl.when(it + 1 < _HB)
            def _():
                fire(it + 1, rows_b, sgb)

            wait_gather(rows_a, sga)

            @pl.when(i > 0)
            def _():
                wait_out(comp_a, soa)

            compact(rows_a, comp_a)
            pltpu.async_copy(comp_a, out_hbm.at[pl.ds(ob + it, 1)], soa)

            @pl.when(it + 2 < _HB)
            def _():
                fire(it + 2, rows_a, sga)

            wait_gather(rows_b, sgb)

            @pl.when(i > 0)
            def _():
                wait_out(comp_b, sob)

            compact(rows_b, comp_b)
            pltpu.async_copy(comp_b, out_hbm.at[pl.ds(ob + it + 1, 1)], sob)
            return carry

        lax.fori_loop(0, _HB // 2, body, 0)
        wait_out(comp_a, soa)
        wait_out(comp_b, sob)


def kernel(x, table):
    table_p = _pad_table(table)
    return _emb_lookup(x.astype(jnp.int32), table_p)
